# baseline (speedup 1.0000x reference)
import numpy as np
import jax
import jax.numpy as jnp
from jax.experimental import pallas as pl

NC = 19
PUR_T = 0.9
CONF_T = 0.9
MOM = 0.99


def _resize_mat():
    R = np.zeros((64, 32), np.float32)
    for o in range(64):
        src = o / 2.0 - 0.25
        k0 = int(np.floor(src))
        f = src - k0
        k0c = min(max(k0, 0), 31)
        k1c = min(max(k0 + 1, 0), 31)
        R[o, k0c] += 1.0 - f
        R[o, k1c] += f
    return R


_R_NP = _resize_mat()


def _resize_tok(R, x):
    y1 = jax.lax.dot_general(R, x, (((1,), (1,)), ((), ())),
                             preferred_element_type=jnp.float32)
    z = jax.lax.dot_general(R, y1, (((1,), (2,)), ((), ())),
                            preferred_element_type=jnp.float32)
    return jnp.transpose(z, (1, 0, 2)).reshape(64 * 64, 256)


def _tstats_body(p_ref, cnt_ref, conf_ref):
    p = p_ref[0]
    conf = p[0]
    lab = jnp.zeros((8, 512), jnp.int32)
    for c in range(1, NC):
        m = p[c] > conf
        conf = jnp.where(m, p[c], conf)
        lab = jnp.where(m, c, lab)
    labm = jnp.where(conf >= CONF_T, lab, -1)
    lab3 = labm.reshape(8, 64, 8)
    cols = []
    for v in range(-1, NC):
        cols.append((lab3 == v).astype(jnp.float32).sum(axis=(0, 2)))
    cnt_ref[0, 0] = jnp.stack(cols, axis=1)
    conf_ref[0, 0, 0] = conf.reshape(8, 64, 8).sum(axis=(0, 2)) * (1.0 / 64.0)


def _tstats(t_probs):
    B = t_probs.shape[0]
    return pl.pallas_call(
        _tstats_body,
        grid=(B, 64),
        in_specs=[pl.BlockSpec((1, NC, 8, 512), lambda b, r: (b, 0, r, 0))],
        out_specs=(
            pl.BlockSpec((1, 1, 64, 20), lambda b, r: (b, r, 0, 0)),
            pl.BlockSpec((1, 1, 1, 64), lambda b, r: (b, r, 0, 0)),
        ),
        out_shape=(
            jax.ShapeDtypeStruct((B, 64, 64, 20), jnp.float32),
            jax.ShapeDtypeStruct((B, 64, 1, 64), jnp.float32),
        ),
    )(t_probs)


def _sstats_body(l_ref, cnt_ref):
    lab3 = l_ref[0].reshape(8, 64, 8)
    cols = []
    for v in range(NC):
        cols.append((lab3 == v).astype(jnp.float32).sum(axis=(0, 2)))
    cnt_ref[0, 0] = jnp.stack(cols, axis=1)


def _sstats(s_label):
    B = s_label.shape[0]
    return pl.pallas_call(
        _sstats_body,
        grid=(B, 64),
        in_specs=[pl.BlockSpec((1, 8, 512), lambda b, r: (b, r, 0))],
        out_specs=pl.BlockSpec((1, 1, 64, NC), lambda b, r: (b, r, 0, 0)),
        out_shape=jax.ShapeDtypeStruct((B, 64, 64, NC), jnp.float32),
    )(s_label)


def _proto_body(r_ref, sf_ref, tf_ref, st_ref,
                stok_ref, ttok_ref, sproto_ref, tproto_ref, d_ref):
    b = pl.program_id(0)

    @pl.when(b == 0)
    def _():
        sproto_ref[...] = jnp.zeros_like(sproto_ref)
        tproto_ref[...] = jnp.zeros_like(tproto_ref)

    R = r_ref[...]
    stok = _resize_tok(R, sf_ref[0])
    ttok = _resize_tok(R, tf_ref[0])
    stok_ref[0] = stok
    ttok_ref[0] = ttok

    iota = jax.lax.broadcasted_iota(jnp.int32, (4096, NC), 1)
    ones = jnp.ones((4096, 128), jnp.float32)
    st = st_ref[0]

    scnt = st[:, 0:NC]
    smax = jnp.max(scnt, axis=1, keepdims=True)
    smode = jnp.argmax(scnt, axis=1, keepdims=True).astype(jnp.int32)
    s_mask = (smax * (1.0 / 64.0)) >= PUR_T
    s_has_any = jnp.any(s_mask)
    oh_s = jnp.where((smode == iota) & s_mask, 1.0, 0.0)
    cnts = jax.lax.dot_general(oh_s, ones, (((0,), (0,)), ((), ())),
                               preferred_element_type=jnp.float32)[:, :1]
    sums = jax.lax.dot_general(oh_s, stok, (((0,), (0,)), ((), ())),
                               preferred_element_type=jnp.float32)
    means = sums / jnp.maximum(cnts, 1.0)
    upd = (cnts > 0.0) & s_has_any
    sp = sproto_ref[...]
    sp = jnp.where(upd, MOM * sp + (1.0 - MOM) * means, sp)
    sproto_ref[...] = sp
    oh_lab = jnp.where(smode == iota, 1.0, 0.0)
    psel = jax.lax.dot_general(oh_lab, sp, (((1,), (0,)), ((), ())),
                               preferred_element_type=jnp.float32)
    diff = stok - psel
    sd = jnp.sqrt(jnp.sum(diff * diff, axis=1, keepdims=True))
    sd = sd * s_mask.astype(jnp.float32)

    tcnt = st[:, NC:NC + 20]
    conf_tok = st[:, NC + 20:NC + 21]
    tmax = jnp.max(tcnt, axis=1, keepdims=True)
    tmidx = jnp.argmax(tcnt, axis=1, keepdims=True).astype(jnp.int32)
    nvalid = 64.0 - st[:, NC:NC + 1]
    purity = jnp.where(tmidx == 0, 0.0, tmax) / jnp.maximum(nvalid, 1.0)
    t_mask = (purity >= PUR_T) & (nvalid > 0.0) & (conf_tok >= CONF_T)
    t_gate = jnp.any(t_mask) & s_has_any
    tmode = tmidx - 1
    tlab = jnp.maximum(tmode, 0)
    oh_t = jnp.where((tmode == iota) & t_mask, 1.0, 0.0)
    cntt = jax.lax.dot_general(oh_t, ones, (((0,), (0,)), ((), ())),
                               preferred_element_type=jnp.float32)[:, :1]
    sumt = jax.lax.dot_general(oh_t, ttok, (((0,), (0,)), ((), ())),
                               preferred_element_type=jnp.float32)
    meant = sumt / jnp.maximum(cntt, 1.0)
    updt = (cntt > 0.0) & t_gate
    tp = tproto_ref[...]
    tp = jnp.where(updt, MOM * tp + (1.0 - MOM) * meant, tp)
    tproto_ref[...] = tp
    oh_tl = jnp.where(tlab == iota, 1.0, 0.0)
    pselt = jax.lax.dot_general(oh_tl, tp, (((1,), (0,)), ((), ())),
                                preferred_element_type=jnp.float32)
    difft = ttok - pselt
    td = jnp.sqrt(jnp.sum(difft * difft, axis=1, keepdims=True))
    td = td * t_mask.astype(jnp.float32)
    d_ref[0] = jnp.concatenate([sd, td], axis=1)


def _proto_stage(R, s_feat, t_feat, stats):
    B = s_feat.shape[0]
    return pl.pallas_call(
        _proto_body,
        grid=(B,),
        in_specs=[
            pl.BlockSpec((64, 32), lambda b: (0, 0)),
            pl.BlockSpec((1, 256, 32, 32), lambda b: (b, 0, 0, 0)),
            pl.BlockSpec((1, 256, 32, 32), lambda b: (b, 0, 0, 0)),
            pl.BlockSpec((1, 4096, 40), lambda b: (b, 0, 0)),
        ],
        out_specs=(
            pl.BlockSpec((1, 4096, 256), lambda b: (b, 0, 0)),
            pl.BlockSpec((1, 4096, 256), lambda b: (b, 0, 0)),
            pl.BlockSpec((NC, 256), lambda b: (0, 0)),
            pl.BlockSpec((NC, 256), lambda b: (0, 0)),
            pl.BlockSpec((1, 4096, 2), lambda b: (b, 0, 0)),
        ),
        out_shape=(
            jax.ShapeDtypeStruct((B, 4096, 256), jnp.float32),
            jax.ShapeDtypeStruct((B, 4096, 256), jnp.float32),
            jax.ShapeDtypeStruct((NC, 256), jnp.float32),
            jax.ShapeDtypeStruct((NC, 256), jnp.float32),
            jax.ShapeDtypeStruct((B, 4096, 2), jnp.float32),
        ),
    )(R, s_feat, t_feat, stats)


@jax.jit
def _run(s_feat, t_feat, s_label, t_probs):
    B = s_feat.shape[0]
    R = jnp.asarray(_R_NP)
    t_cnt, conf_tok = _tstats(t_probs)
    s_cnt = _sstats(s_label)
    stats = jnp.concatenate([
        s_cnt.reshape(B, 4096, NC),
        t_cnt.reshape(B, 4096, 20),
        conf_tok.reshape(B, 4096, 1)], axis=2)
    s_tok, t_tok, s_proto, t_proto, d = _proto_stage(R, s_feat, t_feat, stats)
    return (s_tok, t_tok, s_proto, t_proto, d[:, :, 0], d[:, :, 1])


def kernel(s_feat_map, t_feat_map, s_label_pixel, t_probs_pixel):
    return _run(s_feat_map, t_feat_map,
                s_label_pixel.astype(jnp.int32), t_probs_pixel)


# speedup vs baseline: 1.4395x; 1.4395x over previous
import numpy as np
import jax
import jax.numpy as jnp
from jax.experimental import pallas as pl

NC = 19
PUR_T = 0.9
CONF_T = 0.9
MOM = 0.99


def _resize_mat():
    R = np.zeros((64, 32), np.float32)
    for o in range(64):
        src = o / 2.0 - 0.25
        k0 = int(np.floor(src))
        f = src - k0
        k0c = min(max(k0, 0), 31)
        k1c = min(max(k0 + 1, 0), 31)
        R[o, k0c] += 1.0 - f
        R[o, k1c] += f
    return R


_R_NP = _resize_mat()

_G_NP = np.zeros((64, 512), np.float32)
for _t in range(64):
    _G_NP[_t, _t * 8:(_t + 1) * 8] = 1.0


def _resize_tok(R, x):
    A = jax.lax.dot_general(R, x, (((1,), (2,)), ((), ())),
                            preferred_element_type=jnp.float32)
    B = jax.lax.dot_general(R, A, (((1,), (2,)), ((), ())),
                            preferred_element_type=jnp.float32)
    return B.reshape(64 * 64, 256)


def _tstats_body(g_ref, p_ref, cnt_ref, conf_ref):
    G = g_ref[...]
    p = p_ref[0]
    conf = p[0]
    lab = jnp.zeros((8, 512), jnp.int32)
    for c in range(1, NC):
        m = p[c] > conf
        conf = jnp.where(m, p[c], conf)
        lab = jnp.where(m, c, lab)
    labm = jnp.where(conf >= CONF_T, lab, -1)
    rows = [jnp.sum((labm == v).astype(jnp.float32), axis=0, keepdims=True)
            for v in range(-1, NC)]
    F = jnp.concatenate(rows, axis=0)
    cnt_ref[0, 0] = jax.lax.dot_general(
        G, F, (((1,), (1,)), ((), ())),
        preferred_element_type=jnp.float32)
    crow = jnp.sum(conf, axis=0, keepdims=True)
    conf_ref[0, 0] = jax.lax.dot_general(
        G, crow, (((1,), (1,)), ((), ())),
        preferred_element_type=jnp.float32) * (1.0 / 64.0)


def _tstats(G, t_probs):
    B = t_probs.shape[0]
    return pl.pallas_call(
        _tstats_body,
        grid=(B, 64),
        in_specs=[
            pl.BlockSpec((64, 512), lambda b, r: (0, 0)),
            pl.BlockSpec((1, NC, 8, 512), lambda b, r: (b, 0, r, 0)),
        ],
        out_specs=(
            pl.BlockSpec((1, 1, 64, 20), lambda b, r: (b, r, 0, 0)),
            pl.BlockSpec((1, 1, 64, 1), lambda b, r: (b, r, 0, 0)),
        ),
        out_shape=(
            jax.ShapeDtypeStruct((B, 64, 64, 20), jnp.float32),
            jax.ShapeDtypeStruct((B, 64, 64, 1), jnp.float32),
        ),
    )(G, t_probs)


def _sstats_body(g_ref, l_ref, cnt_ref):
    G = g_ref[...]
    lab = l_ref[0]
    rows = [jnp.sum((lab == v).astype(jnp.float32), axis=0, keepdims=True)
            for v in range(NC)]
    F = jnp.concatenate(rows, axis=0)
    cnt_ref[0, 0] = jax.lax.dot_general(
        G, F, (((1,), (1,)), ((), ())),
        preferred_element_type=jnp.float32)


def _sstats(G, s_label):
    B = s_label.shape[0]
    return pl.pallas_call(
        _sstats_body,
        grid=(B, 64),
        in_specs=[
            pl.BlockSpec((64, 512), lambda b, r: (0, 0)),
            pl.BlockSpec((1, 8, 512), lambda b, r: (b, r, 0)),
        ],
        out_specs=pl.BlockSpec((1, 1, 64, NC), lambda b, r: (b, r, 0, 0)),
        out_shape=jax.ShapeDtypeStruct((B, 64, 64, NC), jnp.float32),
    )(G, s_label)


def _proto_body(r_ref, sf_ref, tf_ref, st_ref,
                stok_ref, ttok_ref, sproto_ref, tproto_ref, d_ref):
    b = pl.program_id(0)

    @pl.when(b == 0)
    def _():
        sproto_ref[...] = jnp.zeros_like(sproto_ref)
        tproto_ref[...] = jnp.zeros_like(tproto_ref)

    R = r_ref[...]
    stok = _resize_tok(R, sf_ref[0])
    ttok = _resize_tok(R, tf_ref[0])
    stok_ref[0] = stok
    ttok_ref[0] = ttok

    iota = jax.lax.broadcasted_iota(jnp.int32, (4096, NC), 1)
    ones = jnp.ones((4096, 128), jnp.float32)
    st = st_ref[0]

    scnt = st[:, 0:NC]
    smax = jnp.max(scnt, axis=1, keepdims=True)
    smode = jnp.argmax(scnt, axis=1, keepdims=True).astype(jnp.int32)
    s_mask = (smax * (1.0 / 64.0)) >= PUR_T
    s_has_any = jnp.any(s_mask)
    oh_s = jnp.where((smode == iota) & s_mask, 1.0, 0.0)
    cnts = jax.lax.dot_general(oh_s, ones, (((0,), (0,)), ((), ())),
                               preferred_element_type=jnp.float32)[:, :1]
    sums = jax.lax.dot_general(oh_s, stok, (((0,), (0,)), ((), ())),
                               preferred_element_type=jnp.float32)
    means = sums / jnp.maximum(cnts, 1.0)
    upd = (cnts > 0.0) & s_has_any
    sp = sproto_ref[...]
    sp = jnp.where(upd, MOM * sp + (1.0 - MOM) * means, sp)
    sproto_ref[...] = sp
    oh_lab = jnp.where(smode == iota, 1.0, 0.0)
    psel = jax.lax.dot_general(oh_lab, sp, (((1,), (0,)), ((), ())),
                               preferred_element_type=jnp.float32)
    diff = stok - psel
    sd = jnp.sqrt(jnp.sum(diff * diff, axis=1, keepdims=True))
    sd = sd * s_mask.astype(jnp.float32)

    tcnt = st[:, NC:NC + 20]
    conf_tok = st[:, NC + 20:NC + 21]
    tmax = jnp.max(tcnt, axis=1, keepdims=True)
    tmidx = jnp.argmax(tcnt, axis=1, keepdims=True).astype(jnp.int32)
    nvalid = 64.0 - st[:, NC:NC + 1]
    purity = jnp.where(tmidx == 0, 0.0, tmax) / jnp.maximum(nvalid, 1.0)
    t_mask = (purity >= PUR_T) & (nvalid > 0.0) & (conf_tok >= CONF_T)
    t_gate = jnp.any(t_mask) & s_has_any
    tmode = tmidx - 1
    tlab = jnp.maximum(tmode, 0)
    oh_t = jnp.where((tmode == iota) & t_mask, 1.0, 0.0)
    cntt = jax.lax.dot_general(oh_t, ones, (((0,), (0,)), ((), ())),
                               preferred_element_type=jnp.float32)[:, :1]
    sumt = jax.lax.dot_general(oh_t, ttok, (((0,), (0,)), ((), ())),
                               preferred_element_type=jnp.float32)
    meant = sumt / jnp.maximum(cntt, 1.0)
    updt = (cntt > 0.0) & t_gate
    tp = tproto_ref[...]
    tp = jnp.where(updt, MOM * tp + (1.0 - MOM) * meant, tp)
    tproto_ref[...] = tp
    oh_tl = jnp.where(tlab == iota, 1.0, 0.0)
    pselt = jax.lax.dot_general(oh_tl, tp, (((1,), (0,)), ((), ())),
                                preferred_element_type=jnp.float32)
    difft = ttok - pselt
    td = jnp.sqrt(jnp.sum(difft * difft, axis=1, keepdims=True))
    td = td * t_mask.astype(jnp.float32)
    d_ref[0] = jnp.concatenate([sd, td], axis=1)


def _proto_stage(R, s_feat, t_feat, stats):
    B = s_feat.shape[0]
    return pl.pallas_call(
        _proto_body,
        grid=(B,),
        in_specs=[
            pl.BlockSpec((64, 32), lambda b: (0, 0)),
            pl.BlockSpec((1, 256, 32, 32), lambda b: (b, 0, 0, 0)),
            pl.BlockSpec((1, 256, 32, 32), lambda b: (b, 0, 0, 0)),
            pl.BlockSpec((1, 4096, 40), lambda b: (b, 0, 0)),
        ],
        out_specs=(
            pl.BlockSpec((1, 4096, 256), lambda b: (b, 0, 0)),
            pl.BlockSpec((1, 4096, 256), lambda b: (b, 0, 0)),
            pl.BlockSpec((NC, 256), lambda b: (0, 0)),
            pl.BlockSpec((NC, 256), lambda b: (0, 0)),
            pl.BlockSpec((1, 4096, 2), lambda b: (b, 0, 0)),
        ),
        out_shape=(
            jax.ShapeDtypeStruct((B, 4096, 256), jnp.float32),
            jax.ShapeDtypeStruct((B, 4096, 256), jnp.float32),
            jax.ShapeDtypeStruct((NC, 256), jnp.float32),
            jax.ShapeDtypeStruct((NC, 256), jnp.float32),
            jax.ShapeDtypeStruct((B, 4096, 2), jnp.float32),
        ),
    )(R, s_feat, t_feat, stats)


@jax.jit
def _run(s_feat, t_feat, s_label, t_probs):
    B = s_feat.shape[0]
    R = jnp.asarray(_R_NP)
    G = jnp.asarray(_G_NP)
    t_cnt, conf_tok = _tstats(G, t_probs)
    s_cnt = _sstats(G, s_label)
    stats = jnp.concatenate([
        s_cnt.reshape(B, 4096, NC),
        t_cnt.reshape(B, 4096, 20),
        conf_tok.reshape(B, 4096, 1)], axis=2)
    s_tok, t_tok, s_proto, t_proto, d = _proto_stage(R, s_feat, t_feat, stats)
    return (s_tok, t_tok, s_proto, t_proto, d[:, :, 0], d[:, :, 1])


def kernel(s_feat_map, t_feat_map, s_label_pixel, t_probs_pixel):
    return _run(s_feat_map, t_feat_map,
                s_label_pixel.astype(jnp.int32), t_probs_pixel)


# speedup vs baseline: 2.1582x; 1.4993x over previous
import numpy as np
import jax
import jax.numpy as jnp
from jax.experimental import pallas as pl

NC = 19
PUR_T = 0.9
CONF_T = 0.9
MOM = 0.99


def _resize_mat():
    R = np.zeros((64, 32), np.float32)
    for o in range(64):
        src = o / 2.0 - 0.25
        k0 = int(np.floor(src))
        f = src - k0
        k0c = min(max(k0, 0), 31)
        k1c = min(max(k0 + 1, 0), 31)
        R[o, k0c] += 1.0 - f
        R[o, k1c] += f
    return R


_R_NP = _resize_mat()

_G_NP = np.zeros((64, 512), np.float32)
for _t in range(64):
    _G_NP[_t, _t * 8:(_t + 1) * 8] = 1.0


def _resize_tok(R, x):
    A = jax.lax.dot_general(R, x, (((1,), (2,)), ((), ())),
                            preferred_element_type=jnp.float32)
    B = jax.lax.dot_general(R, A, (((1,), (2,)), ((), ())),
                            preferred_element_type=jnp.float32)
    return B.reshape(64 * 64, 256)


def _stats_body(g_ref, p_ref, l_ref, st_ref):
    G = g_ref[...]
    slab = l_ref[0]
    p = p_ref[0]
    conf = p[0]
    lab = jnp.zeros((8, 512), jnp.int32)
    for c in range(1, NC):
        m = p[c] > conf
        conf = jnp.where(m, p[c], conf)
        lab = jnp.where(m, c, lab)
    labm = jnp.where(conf >= CONF_T, lab, -1)
    rows = [jnp.sum((slab == v).astype(jnp.float32), axis=0, keepdims=True)
            for v in range(NC)]
    rows += [jnp.sum((labm == v).astype(jnp.float32), axis=0, keepdims=True)
             for v in range(-1, NC)]
    rows.append(jnp.sum(conf, axis=0, keepdims=True))
    F = jnp.concatenate(rows, axis=0)
    st = jax.lax.dot_general(G, F, (((1,), (1,)), ((), ())),
                             preferred_element_type=jnp.float32)
    col = jax.lax.broadcasted_iota(jnp.int32, (1, 40), 1)
    st_ref[0, 0] = st * jnp.where(col == 39, 1.0 / 64.0, 1.0)


def _stats(G, t_probs, s_label):
    B = t_probs.shape[0]
    return pl.pallas_call(
        _stats_body,
        grid=(B, 64),
        in_specs=[
            pl.BlockSpec((64, 512), lambda b, r: (0, 0)),
            pl.BlockSpec((1, NC, 8, 512), lambda b, r: (b, 0, r, 0)),
            pl.BlockSpec((1, 8, 512), lambda b, r: (b, r, 0)),
        ],
        out_specs=pl.BlockSpec((1, 1, 64, 40), lambda b, r: (b, r, 0, 0)),
        out_shape=jax.ShapeDtypeStruct((B, 64, 64, 40), jnp.float32),
    )(G, t_probs, s_label)


def _proto_body(r_ref, sf_ref, tf_ref, st_ref,
                stok_ref, ttok_ref, sproto_ref, tproto_ref, d_ref):
    b = pl.program_id(0)

    @pl.when(b == 0)
    def _():
        sproto_ref[...] = jnp.zeros_like(sproto_ref)
        tproto_ref[...] = jnp.zeros_like(tproto_ref)

    R = r_ref[...]
    stok = _resize_tok(R, sf_ref[0])
    ttok = _resize_tok(R, tf_ref[0])
    stok_ref[0] = stok
    ttok_ref[0] = ttok

    iota = jax.lax.broadcasted_iota(jnp.int32, (4096, NC), 1)
    ones = jnp.ones((4096, 128), jnp.float32)
    st = st_ref[0]

    scnt = st[:, 0:NC]
    smax = jnp.max(scnt, axis=1, keepdims=True)
    smode = jnp.argmax(scnt, axis=1, keepdims=True).astype(jnp.int32)
    s_mask = (smax * (1.0 / 64.0)) >= PUR_T
    s_has_any = jnp.any(s_mask)
    oh_s = jnp.where((smode == iota) & s_mask, 1.0, 0.0)
    cnts = jax.lax.dot_general(oh_s, ones, (((0,), (0,)), ((), ())),
                               preferred_element_type=jnp.float32)[:, :1]
    sums = jax.lax.dot_general(oh_s, stok, (((0,), (0,)), ((), ())),
                               preferred_element_type=jnp.float32)
    means = sums / jnp.maximum(cnts, 1.0)
    upd = (cnts > 0.0) & s_has_any
    sp = sproto_ref[...]
    sp = jnp.where(upd, MOM * sp + (1.0 - MOM) * means, sp)
    sproto_ref[...] = sp
    oh_lab = jnp.where(smode == iota, 1.0, 0.0)
    psel = jax.lax.dot_general(oh_lab, sp, (((1,), (0,)), ((), ())),
                               preferred_element_type=jnp.float32)
    diff = stok - psel
    sd = jnp.sqrt(jnp.sum(diff * diff, axis=1, keepdims=True))
    sd = sd * s_mask.astype(jnp.float32)

    tcnt = st[:, NC:NC + 20]
    conf_tok = st[:, NC + 20:NC + 21]
    tmax = jnp.max(tcnt, axis=1, keepdims=True)
    tmidx = jnp.argmax(tcnt, axis=1, keepdims=True).astype(jnp.int32)
    nvalid = 64.0 - st[:, NC:NC + 1]
    purity = jnp.where(tmidx == 0, 0.0, tmax) / jnp.maximum(nvalid, 1.0)
    t_mask = (purity >= PUR_T) & (nvalid > 0.0) & (conf_tok >= CONF_T)
    t_gate = jnp.any(t_mask) & s_has_any
    tmode = tmidx - 1
    tlab = jnp.maximum(tmode, 0)
    oh_t = jnp.where((tmode == iota) & t_mask, 1.0, 0.0)
    cntt = jax.lax.dot_general(oh_t, ones, (((0,), (0,)), ((), ())),
                               preferred_element_type=jnp.float32)[:, :1]
    sumt = jax.lax.dot_general(oh_t, ttok, (((0,), (0,)), ((), ())),
                               preferred_element_type=jnp.float32)
    meant = sumt / jnp.maximum(cntt, 1.0)
    updt = (cntt > 0.0) & t_gate
    tp = tproto_ref[...]
    tp = jnp.where(updt, MOM * tp + (1.0 - MOM) * meant, tp)
    tproto_ref[...] = tp
    oh_tl = jnp.where(tlab == iota, 1.0, 0.0)
    pselt = jax.lax.dot_general(oh_tl, tp, (((1,), (0,)), ((), ())),
                                preferred_element_type=jnp.float32)
    difft = ttok - pselt
    td = jnp.sqrt(jnp.sum(difft * difft, axis=1, keepdims=True))
    td = td * t_mask.astype(jnp.float32)
    d_ref[0] = jnp.concatenate([sd, td], axis=1)


def _proto_stage(R, s_feat, t_feat, stats):
    B = s_feat.shape[0]
    return pl.pallas_call(
        _proto_body,
        grid=(B,),
        in_specs=[
            pl.BlockSpec((64, 32), lambda b: (0, 0)),
            pl.BlockSpec((1, 256, 32, 32), lambda b: (b, 0, 0, 0)),
            pl.BlockSpec((1, 256, 32, 32), lambda b: (b, 0, 0, 0)),
            pl.BlockSpec((1, 4096, 40), lambda b: (b, 0, 0)),
        ],
        out_specs=(
            pl.BlockSpec((1, 4096, 256), lambda b: (b, 0, 0)),
            pl.BlockSpec((1, 4096, 256), lambda b: (b, 0, 0)),
            pl.BlockSpec((NC, 256), lambda b: (0, 0)),
            pl.BlockSpec((NC, 256), lambda b: (0, 0)),
            pl.BlockSpec((1, 4096, 2), lambda b: (b, 0, 0)),
        ),
        out_shape=(
            jax.ShapeDtypeStruct((B, 4096, 256), jnp.float32),
            jax.ShapeDtypeStruct((B, 4096, 256), jnp.float32),
            jax.ShapeDtypeStruct((NC, 256), jnp.float32),
            jax.ShapeDtypeStruct((NC, 256), jnp.float32),
            jax.ShapeDtypeStruct((B, 4096, 2), jnp.float32),
        ),
    )(R, s_feat, t_feat, stats)


@jax.jit
def _run(s_feat, t_feat, s_label, t_probs):
    B = s_feat.shape[0]
    R = jnp.asarray(_R_NP)
    G = jnp.asarray(_G_NP)
    stats = _stats(G, t_probs, s_label).reshape(B, 4096, 40)
    s_tok, t_tok, s_proto, t_proto, d = _proto_stage(R, s_feat, t_feat, stats)
    return (s_tok, t_tok, s_proto, t_proto, d[:, :, 0], d[:, :, 1])


def kernel(s_feat_map, t_feat_map, s_label_pixel, t_probs_pixel):
    return _run(s_feat_map, t_feat_map,
                s_label_pixel.astype(jnp.int32), t_probs_pixel)
